# baseline (speedup 1.0000x reference)
import jax
import jax.numpy as jnp
from jax.experimental import pallas as pl
from jax.experimental.pallas import tpu as pltpu

N = 4096
D = 64
BM = 512


def _inner(inp_ref, w_ref, out_ref):
    out_ref[...] = jnp.dot(inp_ref[...], w_ref[...],
                           preferred_element_type=jnp.float32)


def _outer(inp_hbm, w_hbm, out_hbm):
    pipeline = pltpu.emit_pipeline(
        _inner,
        grid=(N // BM,),
        in_specs=[
            pl.BlockSpec((BM, N), lambda i: (i, 0)),
            pl.BlockSpec((N, D), lambda i: (0, 0)),
        ],
        out_specs=[pl.BlockSpec((BM, D), lambda i: (i, 0))],
    )
    pipeline(inp_hbm, w_hbm, out_hbm)


@jax.jit
def kernel(inp, weight):
    return pl.pallas_call(
        _outer,
        in_specs=[
            pl.BlockSpec(memory_space=pltpu.MemorySpace.HBM),
            pl.BlockSpec(memory_space=pltpu.MemorySpace.HBM),
        ],
        out_specs=pl.BlockSpec(memory_space=pltpu.MemorySpace.HBM),
        out_shape=jax.ShapeDtypeStruct((N, D), jnp.float32),
        compiler_params=pltpu.CompilerParams(
            skip_device_barrier=True,
            disable_bounds_checks=True,
        ),
    )(inp, weight)
